# baseline (speedup 1.0000x reference)
import functools

import jax
import jax.numpy as jnp
from jax import lax
from jax.experimental import pallas as pl
from jax.experimental.pallas import tpu as pltpu
from jax.experimental.pallas import tpu_sc as plsc

N = 10000
E = 320000
D = 128

NC = 2
NS = 16
NW = NC * NS
CHUNK = 32
PASSES = 4
PER = 80
NCHUNK = PASSES * PER
E_PAD = NW * NCHUNK * CHUNK
N_PAD = 10240
RPT = N_PAD // NS


def _sc_segment_sum(kernel_name, table, src_w, dst_w, zeros):
    mesh = plsc.VectorSubcoreMesh(core_axis_name="c", subcore_axis_name="s")

    @functools.partial(
        pl.kernel,
        out_type=jax.ShapeDtypeStruct((NC, N_PAD, D), jnp.float32),
        mesh=mesh,
        scratch_types=[
            pltpu.VMEM((PER, CHUNK), jnp.int32),
            pltpu.VMEM((PER, CHUNK), jnp.int32),
            pltpu.VMEM((CHUNK, D), jnp.float32),
            pltpu.VMEM_SHARED((N_PAD, D), jnp.float32),
            pltpu.SemaphoreType.DMA,
        ],
        name=kernel_name,
    )
    def k(table_hbm, src_hbm, dst_hbm, zeros_hbm, out_hbm, src_v, dst_v,
          rows_v, agg, sem):
        c = lax.axis_index("c")
        s = lax.axis_index("s")
        wid = s * NC + c
        pltpu.sync_copy(zeros_hbm.at[pl.ds(s * RPT, RPT)],
                        agg.at[pl.ds(s * RPT, RPT)])
        plsc.subcore_barrier()

        def body(j, carry):
            pltpu.async_copy(table_hbm.at[src_v.at[j]], rows_v, sem).wait()
            pltpu.sync_copy(rows_v, agg.at[dst_v.at[j]], add=True)
            return carry

        for p in range(PASSES):
            pltpu.sync_copy(src_hbm.at[wid, pl.ds(p * PER, PER)], src_v)
            pltpu.sync_copy(dst_hbm.at[wid, pl.ds(p * PER, PER)], dst_v)
            lax.fori_loop(0, PER, body, 0)
        plsc.subcore_barrier()
        pltpu.sync_copy(agg.at[pl.ds(s * RPT, RPT)],
                        out_hbm.at[c, pl.ds(s * RPT, RPT)])

    return k(table, src_w, dst_w, zeros)


def _mlp_block(x_ref, a0_ref, a1_ref, wa_ref, ba_ref, wb_ref, bb_ref, o_ref):
    h = x_ref[...] + a0_ref[0] + a1_ref[0]
    h = jnp.dot(h, wa_ref[...], preferred_element_type=jnp.float32)
    h = jnp.maximum(h + ba_ref[...], 0.0)
    h = jnp.dot(h, wb_ref[...], preferred_element_type=jnp.float32)
    h = jnp.maximum(h + bb_ref[...], 0.0)
    o_ref[...] = h[:, : o_ref.shape[1]]


def _mlp(x, agg, wa, ba, wb, bb, dout):
    nb = 10
    rb = N // nb
    row = lambda i: (i, 0)
    full = lambda i: (0, 0)
    return pl.pallas_call(
        _mlp_block,
        grid=(nb,),
        in_specs=[
            pl.BlockSpec((rb, D), row),
            pl.BlockSpec((1, rb, D), lambda i: (0, i, 0)),
            pl.BlockSpec((1, rb, D), lambda i: (1, i, 0)),
            pl.BlockSpec((D, D), full),
            pl.BlockSpec((1, D), full),
            pl.BlockSpec((D, D), full),
            pl.BlockSpec((1, D), full),
        ],
        out_specs=pl.BlockSpec((rb, dout), row),
        out_shape=jax.ShapeDtypeStruct((N, dout), jnp.float32),
    )(x, agg, agg, wa, ba.reshape(1, D), wb, bb.reshape(1, D))


def _edge_slabs(edge_index):
    src = edge_index[0].astype(jnp.int32)
    dst = edge_index[1].astype(jnp.int32)
    pad = E_PAD - E
    src_w = jnp.concatenate(
        [src, jnp.zeros((pad,), jnp.int32)]).reshape(NW, NCHUNK, CHUNK)
    dst_w = jnp.concatenate(
        [dst, jnp.full((pad,), N, jnp.int32)]).reshape(NW, NCHUNK, CHUNK)
    return src_w, dst_w


def kernel(x, edge_index, W1a, b1a, W1b, b1b, W2a, b2a, W2b, b2b):
    src_w, dst_w = _edge_slabs(edge_index)
    zeros = jnp.zeros((N_PAD, D), jnp.float32)

    agg1 = _sc_segment_sum("gin_agg1", x, src_w, dst_w, zeros)
    h1 = _mlp(x, agg1, W1a, b1a, W1b, b1b, D)

    W2a_p = jnp.zeros((D, D), jnp.float32).at[:, :64].set(W2a)
    b2a_p = jnp.zeros((D,), jnp.float32).at[:64].set(b2a)
    W2b_p = jnp.zeros((D, D), jnp.float32).at[:64, :64].set(W2b)
    b2b_p = jnp.zeros((D,), jnp.float32).at[:64].set(b2b)

    agg2 = _sc_segment_sum("gin_agg2", h1, src_w, dst_w, zeros)
    return _mlp(h1, agg2, W2a_p, b2a_p, W2b_p, b2b_p, 64)


# speedup vs baseline: 2.0769x; 2.0769x over previous
import functools

import jax
import jax.numpy as jnp
from jax import lax
from jax.experimental import pallas as pl
from jax.experimental.pallas import tpu as pltpu
from jax.experimental.pallas import tpu_sc as plsc

N = 10000
E = 320000
D = 128

NC = 2
NS = 16
NW = NC * NS
CHUNK = 96
PASSES = 1
PER = 105
NCHUNK = PASSES * PER
E_PAD = NW * NCHUNK * CHUNK
N_PAD = 10240
RPT = N_PAD // NS


def _sc_segment_sum(kernel_name, table, src_w, dst_w, zeros):
    mesh = plsc.VectorSubcoreMesh(core_axis_name="c", subcore_axis_name="s")

    @functools.partial(
        pl.kernel,
        out_type=jax.ShapeDtypeStruct((NC, N_PAD, D), jnp.float32),
        mesh=mesh,
        scratch_types=[
            pltpu.VMEM((PER, CHUNK), jnp.int32),
            pltpu.VMEM((PER, CHUNK), jnp.int32),
            pltpu.VMEM((CHUNK, D), jnp.float32),
            pltpu.VMEM_SHARED((N_PAD, D), jnp.float32),
            pltpu.SemaphoreType.DMA,
        ],
        name=kernel_name,
    )
    def k(table_hbm, src_hbm, dst_hbm, zeros_hbm, out_hbm, src_v, dst_v,
          rows_v, agg, sem):
        c = lax.axis_index("c")
        s = lax.axis_index("s")
        wid = s * NC + c
        pltpu.sync_copy(zeros_hbm.at[pl.ds(s * RPT, RPT)],
                        agg.at[pl.ds(s * RPT, RPT)])
        plsc.subcore_barrier()

        def body(j, carry):
            pltpu.async_copy(table_hbm.at[src_v.at[j]], rows_v, sem).wait()
            pltpu.sync_copy(rows_v, agg.at[dst_v.at[j]], add=True)
            return carry

        pltpu.sync_copy(src_hbm.at[wid], src_v)
        pltpu.sync_copy(dst_hbm.at[wid], dst_v)
        lax.fori_loop(0, NCHUNK, body, 0)
        plsc.subcore_barrier()
        pltpu.sync_copy(agg.at[pl.ds(s * RPT, RPT)],
                        out_hbm.at[c, pl.ds(s * RPT, RPT)])

    return k(table, src_w, dst_w, zeros)


def _mlp_block(x_ref, a0_ref, a1_ref, wa_ref, ba_ref, wb_ref, bb_ref, o_ref):
    h = x_ref[...] + a0_ref[0] + a1_ref[0]
    h = jnp.dot(h, wa_ref[...], preferred_element_type=jnp.float32)
    h = jnp.maximum(h + ba_ref[...], 0.0)
    h = jnp.dot(h, wb_ref[...], preferred_element_type=jnp.float32)
    h = jnp.maximum(h + bb_ref[...], 0.0)
    o_ref[...] = h[:, : o_ref.shape[1]]


def _mlp(x, agg, wa, ba, wb, bb, dout):
    nb = 10
    rb = N // nb
    row = lambda i: (i, 0)
    full = lambda i: (0, 0)
    return pl.pallas_call(
        _mlp_block,
        grid=(nb,),
        in_specs=[
            pl.BlockSpec((rb, D), row),
            pl.BlockSpec((1, rb, D), lambda i: (0, i, 0)),
            pl.BlockSpec((1, rb, D), lambda i: (1, i, 0)),
            pl.BlockSpec((D, D), full),
            pl.BlockSpec((1, D), full),
            pl.BlockSpec((D, D), full),
            pl.BlockSpec((1, D), full),
        ],
        out_specs=pl.BlockSpec((rb, dout), row),
        out_shape=jax.ShapeDtypeStruct((N, dout), jnp.float32),
    )(x, agg, agg, wa, ba.reshape(1, D), wb, bb.reshape(1, D))


def _edge_slabs(edge_index):
    src = edge_index[0].astype(jnp.int32)
    dst = edge_index[1].astype(jnp.int32)
    pad = E_PAD - E
    src_w = jnp.concatenate(
        [src, jnp.zeros((pad,), jnp.int32)]).reshape(NW, NCHUNK, CHUNK)
    dst_w = jnp.concatenate(
        [dst, jnp.full((pad,), N, jnp.int32)]).reshape(NW, NCHUNK, CHUNK)
    return src_w, dst_w


def kernel(x, edge_index, W1a, b1a, W1b, b1b, W2a, b2a, W2b, b2b):
    src_w, dst_w = _edge_slabs(edge_index)
    zeros = jnp.zeros((N_PAD, D), jnp.float32)

    agg1 = _sc_segment_sum("gin_agg1", x, src_w, dst_w, zeros)
    h1 = _mlp(x, agg1, W1a, b1a, W1b, b1b, D)

    W2a_p = jnp.zeros((D, D), jnp.float32).at[:, :64].set(W2a)
    b2a_p = jnp.zeros((D,), jnp.float32).at[:64].set(b2a)
    W2b_p = jnp.zeros((D, D), jnp.float32).at[:64, :64].set(W2b)
    b2b_p = jnp.zeros((D,), jnp.float32).at[:64].set(b2b)

    agg2 = _sc_segment_sum("gin_agg2", h1, src_w, dst_w, zeros)
    return _mlp(h1, agg2, W2a_p, b2a_p, W2b_p, b2b_p, 64)


# speedup vs baseline: 2.1413x; 1.0310x over previous
import functools

import jax
import jax.numpy as jnp
from jax import lax
from jax.experimental import pallas as pl
from jax.experimental.pallas import tpu as pltpu
from jax.experimental.pallas import tpu_sc as plsc

N = 10000
E = 320000
D = 128

NC = 2
NS = 16
NW = NC * NS
CHUNK = 64
NCHUNK = -(-E // (NW * CHUNK))
E_PAD = NW * NCHUNK * CHUNK
N_PAD = 10240
RPT = N_PAD // NS


def _sc_segment_sum(kernel_name, table, src_w, dst_w, zeros):
    mesh = plsc.VectorSubcoreMesh(core_axis_name="c", subcore_axis_name="s")

    @functools.partial(
        pl.kernel,
        out_type=jax.ShapeDtypeStruct((NC, N_PAD, D), jnp.float32),
        mesh=mesh,
        scratch_types=[
            pltpu.VMEM((NCHUNK, CHUNK), jnp.int32),
            pltpu.VMEM((NCHUNK, CHUNK), jnp.int32),
            pltpu.VMEM((CHUNK, D), jnp.float32),
            pltpu.VMEM_SHARED((N_PAD, D), jnp.float32),
            pltpu.SemaphoreType.DMA,
        ],
        name=kernel_name,
    )
    def k(table_hbm, src_hbm, dst_hbm, zeros_hbm, out_hbm, src_v, dst_v,
          rows_v, agg, sem):
        c = lax.axis_index("c")
        s = lax.axis_index("s")
        wid = s * NC + c
        pltpu.sync_copy(zeros_hbm.at[pl.ds(s * RPT, RPT)],
                        agg.at[pl.ds(s * RPT, RPT)])
        pltpu.sync_copy(src_hbm.at[wid], src_v)
        pltpu.sync_copy(dst_hbm.at[wid], dst_v)
        plsc.subcore_barrier()

        def body(j, carry):
            pltpu.async_copy(table_hbm.at[src_v.at[j]], rows_v, sem).wait()
            pltpu.sync_copy(rows_v, agg.at[dst_v.at[j]], add=True)
            return carry

        lax.fori_loop(0, NCHUNK, body, 0)
        plsc.subcore_barrier()
        pltpu.sync_copy(agg.at[pl.ds(s * RPT, RPT)],
                        out_hbm.at[c, pl.ds(s * RPT, RPT)])

    return k(table, src_w, dst_w, zeros)


def _mlp_block(x_ref, a0_ref, a1_ref, wa_ref, ba_ref, wb_ref, bb_ref, o_ref):
    h = x_ref[...] + a0_ref[0] + a1_ref[0]
    h = jnp.dot(h, wa_ref[...], preferred_element_type=jnp.float32)
    h = jnp.maximum(h + ba_ref[...], 0.0)
    h = jnp.dot(h, wb_ref[...], preferred_element_type=jnp.float32)
    h = jnp.maximum(h + bb_ref[...], 0.0)
    o_ref[...] = h[:, : o_ref.shape[1]]


def _mlp(x, agg, wa, ba, wb, bb, dout):
    nb = 10
    rb = N // nb
    row = lambda i: (i, 0)
    full = lambda i: (0, 0)
    return pl.pallas_call(
        _mlp_block,
        grid=(nb,),
        in_specs=[
            pl.BlockSpec((rb, D), row),
            pl.BlockSpec((1, rb, D), lambda i: (0, i, 0)),
            pl.BlockSpec((1, rb, D), lambda i: (1, i, 0)),
            pl.BlockSpec((D, D), full),
            pl.BlockSpec((1, D), full),
            pl.BlockSpec((D, D), full),
            pl.BlockSpec((1, D), full),
        ],
        out_specs=pl.BlockSpec((rb, dout), row),
        out_shape=jax.ShapeDtypeStruct((N, dout), jnp.float32),
    )(x, agg, agg, wa, ba.reshape(1, D), wb, bb.reshape(1, D))


def _edge_slabs(edge_index):
    src = edge_index[0].astype(jnp.int32)
    dst = edge_index[1].astype(jnp.int32)
    pad = E_PAD - E
    src_w = jnp.concatenate(
        [src, jnp.zeros((pad,), jnp.int32)]).reshape(NW, NCHUNK, CHUNK)
    dst_w = jnp.concatenate(
        [dst, jnp.full((pad,), N, jnp.int32)]).reshape(NW, NCHUNK, CHUNK)
    return src_w, dst_w


def kernel(x, edge_index, W1a, b1a, W1b, b1b, W2a, b2a, W2b, b2b):
    src_w, dst_w = _edge_slabs(edge_index)
    zeros = jnp.zeros((N_PAD, D), jnp.float32)

    agg1 = _sc_segment_sum("gin_agg1", x, src_w, dst_w, zeros)
    h1 = _mlp(x, agg1, W1a, b1a, W1b, b1b, D)

    W2a_p = jnp.zeros((D, D), jnp.float32).at[:, :64].set(W2a)
    b2a_p = jnp.zeros((D,), jnp.float32).at[:64].set(b2a)
    W2b_p = jnp.zeros((D, D), jnp.float32).at[:64, :64].set(W2b)
    b2b_p = jnp.zeros((D,), jnp.float32).at[:64].set(b2b)

    agg2 = _sc_segment_sum("gin_agg2", h1, src_w, dst_w, zeros)
    return _mlp(h1, agg2, W2a_p, b2a_p, W2b_p, b2b_p, 64)
